# baseline (speedup 1.0000x reference)
import functools

import jax
import jax.numpy as jnp
from jax import lax
from jax.experimental import pallas as pl
from jax.experimental.pallas import tpu as pltpu
from jax.experimental.pallas import tpu_sc as plsc

_EPS = 1e-07
_W = 64
_S = 100
_NW = 32
_BPW = 128
_NV = _BPW // 16
_LN2 = 0.6931471805599453


def _prep_body(bwt_ref, mw_ref, dt_ref, aux_ref):
    bwt = bwt_ref[...]
    p = jax.nn.sigmoid(bwt)
    a = jnp.log(p + _EPS)
    c = jnp.log(1.0 - p + _EPS)
    u = 0.5 * jnp.sum(a + c, axis=0)
    mw = mw_ref[0, :]
    mixp = jnp.exp(mw - jnp.max(mw))
    mixp = mixp / jnp.sum(mixp)
    umax = jnp.max(u)
    coef = mixp * jnp.exp(u - umax)
    dt_ref[...] = 0.5 * (a - c)
    aux_ref[0:_W] = coef
    aux_ref[_W:2 * _W] = jnp.full((_W,), umax, jnp.float32)


def _bcast_lane(v, idx):
    return v.at[idx].get(mode=lax.GatherScatterMode.PROMISE_IN_BOUNDS)


def _log16(x):
    xi = lax.bitcast_convert_type(x, jnp.int32)
    e = lax.shift_right_logical(xi, 23) - 127
    m = lax.bitcast_convert_type((xi & 0x007FFFFF) | 0x3F800000,
                                 jnp.float32)
    big = m > (4.0 / 3.0)
    m = jnp.where(big, 0.5 * m, m)
    e = e + jnp.where(big, 1, 0)
    z = m - 1.0
    pz = jnp.float32(-0.125)
    for kk in (7, 6, 5, 4, 3, 2):
        pz = pz * z + ((1.0 / kk) if kk % 2 else (-1.0 / kk))
    pz = z * (1.0 + z * pz)
    return e.astype(jnp.float32) * _LN2 + pz


def _sc_body(st_hbm, dt_hbm, aux_hbm, out_hbm, st_v, dt_v, aux_v, out_v):
    wid = lax.axis_index("s") * 2 + lax.axis_index("c")
    pltpu.sync_copy(st_hbm.at[:, pl.ds(wid * _BPW, _BPW)], st_v)
    pltpu.sync_copy(dt_hbm, dt_v)
    pltpu.sync_copy(aux_hbm, aux_v)

    uvec = aux_v[pl.ds(_W, 16)]
    zero = jnp.zeros((16,), jnp.float32)
    lanes16 = lax.iota(jnp.int32, 16)

    def wt_body(wt, esums):
        g16 = (wt // 4) * 16
        lb = (wt % 4) * 4
        idxs = [(lanes16 & 0) + (lb + j) for j in range(4)]
        cf = aux_v[pl.ds(g16, 16)]

        def site_body(ij, accs):
            accs = list(accs)
            dv = dt_v[ij, pl.ds(g16, 16)]
            dbs = [_bcast_lane(dv, idxs[j]) for j in range(4)]
            svs = [st_v[ij, pl.ds(16 * v, 16)] for v in range(_NV)]
            for v in range(_NV):
                for j in range(4):
                    accs[v * 4 + j] = accs[v * 4 + j] + dbs[j] * svs[v]
            return tuple(accs)

        accs = lax.fori_loop(0, _S, site_body, (zero,) * (_NV * 4))
        esums = list(esums)
        for j in range(4):
            cb = _bcast_lane(cf, idxs[j])
            for v in range(_NV):
                t = accs[v * 4 + j]
                esums[v] = esums[v] + cb * (jnp.exp(t) + jnp.exp(-t))
        return tuple(esums)

    esums = lax.fori_loop(0, _W // 4, wt_body, (zero,) * _NV)
    for v in range(_NV):
        out_v[pl.ds(16 * v, 16)] = _log16(0.5 * esums[v]) + uvec
    pltpu.sync_copy(out_v, out_hbm.at[pl.ds(wid * _BPW, _BPW)])


def kernel(sample, ber_weight, mix_weight):
    b = sample.shape[0]
    st = jnp.transpose(sample, (1, 2, 3, 0)).reshape(_S, b)
    bwt = ber_weight.reshape(_W, _S).T
    mw2 = mix_weight.reshape(1, _W)

    dt, aux = pl.pallas_call(
        _prep_body,
        out_shape=(jax.ShapeDtypeStruct((_S, _W), jnp.float32),
                   jax.ShapeDtypeStruct((2 * _W,), jnp.float32)),
    )(bwt, mw2)

    sc_main = functools.partial(
        pl.kernel,
        out_type=jax.ShapeDtypeStruct((b,), jnp.float32),
        mesh=plsc.VectorSubcoreMesh(core_axis_name="c", subcore_axis_name="s"),
        scratch_types=[
            pltpu.VMEM((_S, _BPW), jnp.float32),
            pltpu.VMEM((_S, _W), jnp.float32),
            pltpu.VMEM((2 * _W,), jnp.float32),
            pltpu.VMEM((_BPW,), jnp.float32),
        ],
    )(_sc_body)
    return sc_main(st, dt, aux)


# speedup vs baseline: 1.3235x; 1.3235x over previous
import functools

import jax
import jax.numpy as jnp
from jax import lax
from jax.experimental import pallas as pl
from jax.experimental.pallas import tpu as pltpu
from jax.experimental.pallas import tpu_sc as plsc

_EPS = 1e-07
_W = 64
_S = 100
_NW = 32
_F = 1024
_BPW = _F // _NW
_NV = _BPW // 16
_LN2 = 0.6931471805599453


def _prep_body(bwt_ref, mw_ref, dt_ref, aux_ref):
    bwt = bwt_ref[...]
    p = jax.nn.sigmoid(bwt)
    a = jnp.log(p + _EPS)
    c = jnp.log(1.0 - p + _EPS)
    u = 0.5 * jnp.sum(a + c, axis=0)
    mw = mw_ref[0, :]
    mixp = jnp.exp(mw - jnp.max(mw))
    mixp = mixp / jnp.sum(mixp)
    umax = jnp.max(u)
    coef = mixp * jnp.exp(u - umax)
    dt_ref[...] = 0.5 * (a - c)
    aux_ref[0:_W] = coef
    aux_ref[_W:2 * _W] = jnp.full((_W,), umax, jnp.float32)


def _bcast_lane(v, idx):
    return v.at[idx].get(mode=lax.GatherScatterMode.PROMISE_IN_BOUNDS)


def _log16(x):
    xi = lax.bitcast_convert_type(x, jnp.int32)
    e = lax.shift_right_logical(xi, 23) - 127
    m = lax.bitcast_convert_type((xi & 0x007FFFFF) | 0x3F800000,
                                 jnp.float32)
    big = m > (4.0 / 3.0)
    m = jnp.where(big, 0.5 * m, m)
    e = e + jnp.where(big, 1, 0)
    z = m - 1.0
    pz = jnp.float32(-0.125)
    for kk in (7, 6, 5, 4, 3, 2):
        pz = pz * z + ((1.0 / kk) if kk % 2 else (-1.0 / kk))
    pz = z * (1.0 + z * pz)
    return e.astype(jnp.float32) * _LN2 + pz


def _sc_body(st_hbm, dt_hbm, aux_hbm, out_hbm, st_v, dt_v, aux_v, out_v):
    wid = lax.axis_index("s") * 2 + lax.axis_index("c")
    cb = wid % (_F // 128)
    q = wid // (_F // 128)
    pltpu.sync_copy(st_hbm.at[:, pl.ds(cb * 128, 128)], st_v)
    pltpu.sync_copy(dt_hbm, dt_v)
    pltpu.sync_copy(aux_hbm, aux_v)

    uvec = aux_v[pl.ds(_W, 16)]
    zero = jnp.zeros((16,), jnp.float32)
    lanes16 = lax.iota(jnp.int32, 16)

    def wt_body(wt, esums):
        g16 = (wt // 4) * 16
        lb = (wt % 4) * 4
        idxs = [(lanes16 & 0) + (lb + j) for j in range(4)]
        cf = aux_v[pl.ds(g16, 16)]

        def site_body(ij, accs):
            accs = list(accs)
            dv = dt_v[ij, pl.ds(g16, 16)]
            dbs = [_bcast_lane(dv, idxs[j]) for j in range(4)]
            svs = [st_v[ij, pl.ds(q * _BPW + 16 * v, 16)] for v in range(_NV)]
            for v in range(_NV):
                for j in range(4):
                    accs[v * 4 + j] = accs[v * 4 + j] + dbs[j] * svs[v]
            return tuple(accs)

        accs = lax.fori_loop(0, _S, site_body, (zero,) * (_NV * 4))
        esums = list(esums)
        for j in range(4):
            cb = _bcast_lane(cf, idxs[j])
            for v in range(_NV):
                t = accs[v * 4 + j]
                esums[v] = esums[v] + cb * (jnp.exp(t) + jnp.exp(-t))
        return tuple(esums)

    esums = lax.fori_loop(0, _W // 4, wt_body, (zero,) * _NV)
    for v in range(_NV):
        out_v[pl.ds(16 * v, 16)] = _log16(0.5 * esums[v]) + uvec
    pltpu.sync_copy(out_v, out_hbm.at[pl.ds(cb * 128 + q * _BPW, _BPW)])


def _tc_body(st_ref, dt_ref, aux_ref, o_ref):
    t = lax.dot_general(
        st_ref[...], dt_ref[...], (((0,), (0,)), ((), ())),
        preferred_element_type=jnp.float32)
    tt = t[_F:, :]
    coef = aux_ref[0:_W]
    umax = jnp.max(aux_ref[_W:2 * _W])
    e = jnp.exp(tt)
    acc = jnp.sum(coef[None, :] * (e + 1.0 / e), axis=1)
    o_ref[...] = jnp.log(0.5 * acc) + umax


def kernel(sample, ber_weight, mix_weight):
    b = sample.shape[0]
    st = jnp.transpose(sample, (1, 2, 3, 0)).reshape(_S, b)
    bwt = ber_weight.reshape(_W, _S).T
    mw2 = mix_weight.reshape(1, _W)

    dt, aux = pl.pallas_call(
        _prep_body,
        out_shape=(jax.ShapeDtypeStruct((_S, _W), jnp.float32),
                   jax.ShapeDtypeStruct((2 * _W,), jnp.float32)),
    )(bwt, mw2)

    sc_main = functools.partial(
        pl.kernel,
        out_type=jax.ShapeDtypeStruct((_F,), jnp.float32),
        mesh=plsc.VectorSubcoreMesh(core_axis_name="c", subcore_axis_name="s"),
        scratch_types=[
            pltpu.VMEM((_S, 128), jnp.float32),
            pltpu.VMEM((_S, _W), jnp.float32),
            pltpu.VMEM((2 * _W,), jnp.float32),
            pltpu.VMEM((_BPW,), jnp.float32),
        ],
    )(_sc_body)
    sc_out = sc_main(st, dt, aux)

    tc_out = pl.pallas_call(
        _tc_body,
        out_shape=jax.ShapeDtypeStruct((b - _F,), jnp.float32),
    )(st, dt, aux)
    return jnp.concatenate([sc_out, tc_out])


# speedup vs baseline: 1.5455x; 1.1677x over previous
import functools

import jax
import jax.numpy as jnp
from jax import lax
from jax.experimental import pallas as pl
from jax.experimental.pallas import tpu as pltpu
from jax.experimental.pallas import tpu_sc as plsc

_EPS = 1e-07
_W = 64
_S = 100
_NW = 32
_F = 1024
_BPW = _F // _NW
_NV = _BPW // 16
_LN2 = 0.6931471805599453


def _prep_body(bwt_ref, mw_ref, dt_ref, aux_ref):
    bwt = bwt_ref[...]
    p = jax.nn.sigmoid(bwt)
    a = jnp.log(p + _EPS)
    c = jnp.log(1.0 - p + _EPS)
    u = 0.5 * jnp.sum(a + c, axis=0)
    mw = mw_ref[0, :]
    mixp = jnp.exp(mw - jnp.max(mw))
    mixp = mixp / jnp.sum(mixp)
    umax = jnp.max(u)
    coef = mixp * jnp.exp(u - umax)
    dt_ref[...] = 0.5 * (a - c)
    aux_ref[0:_W] = coef
    aux_ref[_W:2 * _W] = jnp.full((_W,), umax, jnp.float32)


def _bcast_lane(v, idx):
    return v.at[idx].get(mode=lax.GatherScatterMode.PROMISE_IN_BOUNDS)


def _log16(x):
    xi = lax.bitcast_convert_type(x, jnp.int32)
    e = lax.shift_right_logical(xi, 23) - 127
    m = lax.bitcast_convert_type((xi & 0x007FFFFF) | 0x3F800000,
                                 jnp.float32)
    big = m > (4.0 / 3.0)
    m = jnp.where(big, 0.5 * m, m)
    e = e + jnp.where(big, 1, 0)
    z = m - 1.0
    pz = jnp.float32(-0.125)
    for kk in (7, 6, 5, 4, 3, 2):
        pz = pz * z + ((1.0 / kk) if kk % 2 else (-1.0 / kk))
    pz = z * (1.0 + z * pz)
    return e.astype(jnp.float32) * _LN2 + pz


def _sc_body(st_hbm, dt_hbm, aux_hbm, out_hbm, st_v, dt_v, aux_v, out_v):
    wid = lax.axis_index("s") * 2 + lax.axis_index("c")
    cb = wid % (_F // 128)
    q = wid // (_F // 128)
    pltpu.sync_copy(st_hbm.at[:, pl.ds(cb * 128, 128)], st_v)
    pltpu.sync_copy(dt_hbm, dt_v)
    pltpu.sync_copy(aux_hbm, aux_v)

    uvec = aux_v[pl.ds(_W, 16)]
    zero = jnp.zeros((16,), jnp.float32)
    lanes16 = lax.iota(jnp.int32, 16)

    def wt_body(wt, esums):
        g16 = (wt // 4) * 16
        lb = (wt % 4) * 4
        idxs = [(lanes16 & 0) + (lb + j) for j in range(4)]
        cf = aux_v[pl.ds(g16, 16)]

        def site_body(ij, accs):
            accs = list(accs)
            dv = dt_v[ij, pl.ds(g16, 16)]
            dbs = [_bcast_lane(dv, idxs[j]) for j in range(4)]
            svs = [st_v[ij, pl.ds(q * _BPW + 16 * v, 16)] for v in range(_NV)]
            for v in range(_NV):
                for j in range(4):
                    accs[v * 4 + j] = accs[v * 4 + j] + dbs[j] * svs[v]
            return tuple(accs)

        accs = lax.fori_loop(0, _S, site_body, (zero,) * (_NV * 4))
        esums = list(esums)
        for j in range(4):
            cb = _bcast_lane(cf, idxs[j])
            for v in range(_NV):
                t = accs[v * 4 + j]
                esums[v] = esums[v] + cb * (jnp.exp(t) + jnp.exp(-t))
        return tuple(esums)

    esums = lax.fori_loop(0, _W // 4, wt_body, (zero,) * _NV)
    for v in range(_NV):
        out_v[pl.ds(16 * v, 16)] = _log16(0.5 * esums[v]) + uvec
    pltpu.sync_copy(out_v, out_hbm.at[pl.ds(cb * 128 + q * _BPW, _BPW)])


def _tc_body(st_ref, dt_ref, aux_ref, o_ref):
    t = lax.dot_general(
        st_ref[...], dt_ref[...], (((0,), (0,)), ((), ())),
        preferred_element_type=jnp.float32)
    tt = t[_F:, :]
    coef = aux_ref[0:_W]
    umax = jnp.max(aux_ref[_W:2 * _W])
    e = jnp.exp(tt)
    acc = jnp.sum(coef[None, :] * (e + 1.0 / e), axis=1)
    o_ref[...] = jnp.log(0.5 * acc) + umax


def kernel(sample, ber_weight, mix_weight):
    b = sample.shape[0]
    st = jnp.transpose(sample, (1, 2, 3, 0)).reshape(_S, b)
    bwt = ber_weight.reshape(_W, _S).T
    mw2 = mix_weight.reshape(1, _W)

    dt, aux = pl.pallas_call(
        _prep_body,
        out_shape=(jax.ShapeDtypeStruct((_S, _W), jnp.float32),
                   jax.ShapeDtypeStruct((2 * _W,), jnp.float32)),
    )(bwt, mw2)

    sc_main = functools.partial(
        pl.kernel,
        out_type=jax.ShapeDtypeStruct((_F,), jnp.float32),
        mesh=plsc.VectorSubcoreMesh(core_axis_name="c", subcore_axis_name="s"),
        compiler_params=pltpu.CompilerParams(use_tc_tiling_on_sc=False),
        scratch_types=[
            pltpu.VMEM((_S, 128), jnp.float32),
            pltpu.VMEM((_S, _W), jnp.float32),
            pltpu.VMEM((2 * _W,), jnp.float32),
            pltpu.VMEM((_BPW,), jnp.float32),
        ],
    )(_sc_body)
    sc_out = sc_main(st, dt, aux)

    tc_out = pl.pallas_call(
        _tc_body,
        out_shape=jax.ShapeDtypeStruct((b - _F,), jnp.float32),
    )(st, dt, aux)
    return jnp.concatenate([sc_out, tc_out])


# speedup vs baseline: 1.5747x; 1.0189x over previous
import functools

import jax
import jax.numpy as jnp
from jax import lax
from jax.experimental import pallas as pl
from jax.experimental.pallas import tpu as pltpu
from jax.experimental.pallas import tpu_sc as plsc

_EPS = 1e-07
_W = 64
_S = 100
_NW = 32
_F = 512
_BPW = _F // _NW
_NV = _BPW // 16
_LN2 = 0.6931471805599453


def _prep_body(bwt_ref, mw_ref, dt_ref, aux_ref):
    bwt = bwt_ref[...]
    p = jax.nn.sigmoid(bwt)
    a = jnp.log(p + _EPS)
    c = jnp.log(1.0 - p + _EPS)
    u = 0.5 * jnp.sum(a + c, axis=0)
    mw = mw_ref[0, :]
    mixp = jnp.exp(mw - jnp.max(mw))
    mixp = mixp / jnp.sum(mixp)
    umax = jnp.max(u)
    coef = mixp * jnp.exp(u - umax)
    dt_ref[...] = 0.5 * (a - c)
    aux_ref[0:_W] = coef
    aux_ref[_W:2 * _W] = jnp.full((_W,), umax, jnp.float32)


def _bcast_lane(v, idx):
    return v.at[idx].get(mode=lax.GatherScatterMode.PROMISE_IN_BOUNDS)


def _log16(x):
    xi = lax.bitcast_convert_type(x, jnp.int32)
    e = lax.shift_right_logical(xi, 23) - 127
    m = lax.bitcast_convert_type((xi & 0x007FFFFF) | 0x3F800000,
                                 jnp.float32)
    big = m > (4.0 / 3.0)
    m = jnp.where(big, 0.5 * m, m)
    e = e + jnp.where(big, 1, 0)
    z = m - 1.0
    pz = jnp.float32(-0.125)
    for kk in (7, 6, 5, 4, 3, 2):
        pz = pz * z + ((1.0 / kk) if kk % 2 else (-1.0 / kk))
    pz = z * (1.0 + z * pz)
    return e.astype(jnp.float32) * _LN2 + pz


def _sc_body(st_hbm, dt_hbm, aux_hbm, out_hbm, st_v, dt_v, aux_v, out_v):
    wid = lax.axis_index("s") * 2 + lax.axis_index("c")
    cb = wid % (_F // 128)
    q = wid // (_F // 128)
    pltpu.sync_copy(st_hbm.at[:, pl.ds(cb * 128, 128)], st_v)
    pltpu.sync_copy(dt_hbm, dt_v)
    pltpu.sync_copy(aux_hbm, aux_v)

    uvec = aux_v[pl.ds(_W, 16)]
    zero = jnp.zeros((16,), jnp.float32)
    lanes16 = lax.iota(jnp.int32, 16)

    def wt_body(wt, esums):
        g16 = (wt // 4) * 16
        lb = (wt % 4) * 4
        idxs = [(lanes16 & 0) + (lb + j) for j in range(4)]
        cf = aux_v[pl.ds(g16, 16)]

        def site_body(ij, accs):
            accs = list(accs)
            dv = dt_v[ij, pl.ds(g16, 16)]
            dbs = [_bcast_lane(dv, idxs[j]) for j in range(4)]
            svs = [st_v[ij, pl.ds(q * _BPW + 16 * v, 16)] for v in range(_NV)]
            for v in range(_NV):
                for j in range(4):
                    accs[v * 4 + j] = accs[v * 4 + j] + dbs[j] * svs[v]
            return tuple(accs)

        accs = lax.fori_loop(0, _S, site_body, (zero,) * (_NV * 4))
        esums = list(esums)
        for j in range(4):
            cb = _bcast_lane(cf, idxs[j])
            for v in range(_NV):
                t = accs[v * 4 + j]
                esums[v] = esums[v] + cb * (jnp.exp(t) + jnp.exp(-t))
        return tuple(esums)

    esums = lax.fori_loop(0, _W // 4, wt_body, (zero,) * _NV)
    for v in range(_NV):
        out_v[pl.ds(16 * v, 16)] = _log16(0.5 * esums[v]) + uvec
    pltpu.sync_copy(out_v, out_hbm.at[pl.ds(cb * 128 + q * _BPW, _BPW)])


def _tc_body(st_ref, dt_ref, aux_ref, o_ref):
    t = lax.dot_general(
        st_ref[...], dt_ref[...], (((0,), (0,)), ((), ())),
        preferred_element_type=jnp.float32)
    tt = t[_F:, :]
    coef = aux_ref[0:_W]
    umax = jnp.max(aux_ref[_W:2 * _W])
    e = jnp.exp(tt)
    acc = jnp.sum(coef[None, :] * (e + 1.0 / e), axis=1)
    o_ref[...] = jnp.log(0.5 * acc) + umax


def kernel(sample, ber_weight, mix_weight):
    b = sample.shape[0]
    st = jnp.transpose(sample, (1, 2, 3, 0)).reshape(_S, b)
    bwt = ber_weight.reshape(_W, _S).T
    mw2 = mix_weight.reshape(1, _W)

    dt, aux = pl.pallas_call(
        _prep_body,
        out_shape=(jax.ShapeDtypeStruct((_S, _W), jnp.float32),
                   jax.ShapeDtypeStruct((2 * _W,), jnp.float32)),
    )(bwt, mw2)

    sc_main = functools.partial(
        pl.kernel,
        out_type=jax.ShapeDtypeStruct((_F,), jnp.float32),
        mesh=plsc.VectorSubcoreMesh(core_axis_name="c", subcore_axis_name="s"),
        compiler_params=pltpu.CompilerParams(use_tc_tiling_on_sc=False),
        scratch_types=[
            pltpu.VMEM((_S, 128), jnp.float32),
            pltpu.VMEM((_S, _W), jnp.float32),
            pltpu.VMEM((2 * _W,), jnp.float32),
            pltpu.VMEM((_BPW,), jnp.float32),
        ],
    )(_sc_body)
    sc_out = sc_main(st, dt, aux)

    tc_out = pl.pallas_call(
        _tc_body,
        out_shape=jax.ShapeDtypeStruct((b - _F,), jnp.float32),
    )(st, dt, aux)
    return jnp.concatenate([sc_out, tc_out])
